# baseline (speedup 1.0000x reference)
import functools

import jax
import jax.numpy as jnp
from jax import lax
from jax.experimental import pallas as pl
from jax.experimental.pallas import tpu as pltpu
from jax.experimental.pallas import tpu_sc as plsc

HIDDEN = 2048
TOOL_HID = 512
MAX_TOOLS = 128
TOOL_EMB = 256
TOP_K = 3

TOKEN_BLOCK = 256

SC_CORES = 2
SC_SUBCORES = 16
SC_WORKERS = SC_CORES * SC_SUBCORES
GATHER_CHUNK = 128
TABLE_REP = 256


def _tc_body(x_ref, wgt_ref, bg_ref, w1_ref, b1_ref, w2_ref, b2_ref,
             enh_ref, probs_ref, gate_ref, idx_ref, idxs_ref):
    x = x_ref[...]
    enh_ref[...] = x

    xb = x.astype(jnp.bfloat16).astype(jnp.float32)
    wb = wgt_ref[...].astype(jnp.bfloat16).astype(jnp.float32)
    z = jnp.sum(xb * wb, axis=-1, keepdims=True) + bg_ref[...]
    gate_ref[...] = z > 0.0

    h = jnp.maximum(
        jnp.dot(x, w1_ref[...], preferred_element_type=jnp.float32)
        + b1_ref[...], 0.0)
    logits = (jnp.dot(h, w2_ref[...], preferred_element_type=jnp.float32)
              + b2_ref[...])
    m = jnp.max(logits, axis=-1, keepdims=True)
    e = jnp.exp(logits - m)
    probs = e / jnp.sum(e, axis=-1, keepdims=True)
    probs_ref[...] = probs

    iota = lax.broadcasted_iota(jnp.int32, probs.shape, 1)
    p = probs
    cols = []
    for _ in range(TOP_K):
        pm = jnp.max(p, axis=-1, keepdims=True)
        a = jnp.min(jnp.where(p == pm, iota, MAX_TOOLS), axis=-1,
                    keepdims=True)
        cols.append(a)
        p = jnp.where(iota == a, -1.0, p)
    idx = jnp.concatenate(cols, axis=1)
    idx_ref[...] = idx
    r3 = 3 * lax.broadcasted_iota(jnp.int32, idx.shape, 0)
    kk = lax.broadcasted_iota(jnp.int32, idx.shape, 1)
    idxs_ref[...] = idx + MAX_TOOLS * ((r3 + kk) % TABLE_REP)


def _run_tc(x2d, wg_t, b_gate, W_sel1, b_sel1, W_sel2, b_sel2):
    n = x2d.shape[0]
    grid = (n // TOKEN_BLOCK,)
    tok = lambda i: (i, 0)
    rep = lambda i: (0, 0)
    return pl.pallas_call(
        _tc_body,
        grid=grid,
        in_specs=[
            pl.BlockSpec((TOKEN_BLOCK, HIDDEN), tok),
            pl.BlockSpec((1, HIDDEN), rep),
            pl.BlockSpec((1, 1), rep),
            pl.BlockSpec((HIDDEN, TOOL_HID), rep),
            pl.BlockSpec((1, TOOL_HID), rep),
            pl.BlockSpec((TOOL_HID, MAX_TOOLS), rep),
            pl.BlockSpec((1, MAX_TOOLS), rep),
        ],
        out_specs=[
            pl.BlockSpec((TOKEN_BLOCK, HIDDEN), tok),
            pl.BlockSpec((TOKEN_BLOCK, MAX_TOOLS), tok),
            pl.BlockSpec((TOKEN_BLOCK, 1), tok),
            pl.BlockSpec((TOKEN_BLOCK, TOP_K), tok),
            pl.BlockSpec((TOKEN_BLOCK, TOP_K), tok),
        ],
        out_shape=[
            jax.ShapeDtypeStruct((n, HIDDEN), jnp.float32),
            jax.ShapeDtypeStruct((n, MAX_TOOLS), jnp.float32),
            jax.ShapeDtypeStruct((n, 1), jnp.bool_),
            jax.ShapeDtypeStruct((n, TOP_K), jnp.int32),
            jax.ShapeDtypeStruct((n, TOP_K), jnp.int32),
        ],
        compiler_params=pltpu.CompilerParams(
            dimension_semantics=("arbitrary",)),
    )(x2d, wg_t, b_gate, W_sel1, b_sel1, W_sel2, b_sel2)


def _sc_gather(table_lo, table_hi, idx_flat):
    total = idx_flat.shape[0]
    per_worker = total // SC_WORKERS
    chunks = per_worker // GATHER_CHUNK
    mesh = plsc.VectorSubcoreMesh(core_axis_name="c", subcore_axis_name="s")
    half = TOOL_EMB // 2

    @functools.partial(
        pl.kernel,
        out_type=[
            jax.ShapeDtypeStruct((total, half), jnp.float32),
            jax.ShapeDtypeStruct((total, half), jnp.float32),
        ],
        mesh=mesh,
        scratch_types=[
            pltpu.VMEM((per_worker,), jnp.int32),
            pltpu.VMEM((GATHER_CHUNK, half), jnp.float32),
            pltpu.VMEM((GATHER_CHUNK, half), jnp.float32),
            pltpu.VMEM((GATHER_CHUNK, half), jnp.float32),
            pltpu.VMEM((GATHER_CHUNK, half), jnp.float32),
            pltpu.SemaphoreType.DMA,
            pltpu.SemaphoreType.DMA,
            pltpu.SemaphoreType.DMA,
            pltpu.SemaphoreType.DMA,
            pltpu.SemaphoreType.DMA,
            pltpu.SemaphoreType.DMA,
            pltpu.SemaphoreType.DMA,
            pltpu.SemaphoreType.DMA,
        ],
    )
    def gather_kernel(tlo_hbm, thi_hbm, idx_hbm, lo_hbm, hi_hbm,
                      idx_all, lo0, lo1, hi0, hi1,
                      sgl0, sgl1, sgh0, sgh1, ssl0, ssl1, ssh0, ssh1):
        wid = lax.axis_index("s") * SC_CORES + lax.axis_index("c")
        base = wid * per_worker
        pltpu.sync_copy(idx_hbm.at[pl.ds(base, per_worker)], idx_all)
        lob = (lo0, lo1)
        hib = (hi0, hi1)
        sgl = (sgl0, sgl1)
        sgh = (sgh0, sgh1)
        ssl = (ssl0, ssl1)
        ssh = (ssh0, ssh1)
        gl, gh, sl, sh = {}, {}, {}, {}

        def idx_view(i):
            return idx_all.at[pl.ds(i * GATHER_CHUNK, GATHER_CHUNK)]

        gl[0] = pltpu.async_copy(tlo_hbm.at[idx_view(0)], lo0, sgl0)
        gh[0] = pltpu.async_copy(thi_hbm.at[idx_view(0)], hi0, sgh0)
        for i in range(chunks):
            bi = i & 1
            off = pl.ds(base + i * GATHER_CHUNK, GATHER_CHUNK)
            gl[i].wait()
            sl[i] = pltpu.async_copy(lob[bi], lo_hbm.at[off], ssl[bi])
            gh[i].wait()
            sh[i] = pltpu.async_copy(hib[bi], hi_hbm.at[off], ssh[bi])
            ni = i + 1
            if ni < chunks:
                nb = ni & 1
                if ni >= 2:
                    sl[ni - 2].wait()
                    sh[ni - 2].wait()
                gl[ni] = pltpu.async_copy(tlo_hbm.at[idx_view(ni)],
                                          lob[nb], sgl[nb])
                gh[ni] = pltpu.async_copy(thi_hbm.at[idx_view(ni)],
                                          hib[nb], sgh[nb])
        sl[chunks - 2].wait()
        sh[chunks - 2].wait()
        sl[chunks - 1].wait()
        sh[chunks - 1].wait()

    return gather_kernel(table_lo, table_hi, idx_flat)


def _finish_body(lo_ref, hi_ref, out_ref):
    t = out_ref.shape[0]
    lo = lo_ref[...].reshape(t, TOP_K, TOOL_EMB // 2)
    hi = hi_ref[...].reshape(t, TOP_K, TOOL_EMB // 2)
    out_ref[...] = jnp.concatenate([lo, hi], axis=-1)


def _tc_finish(lo, hi):
    total = lo.shape[0]
    n = total // TOP_K
    t = TOKEN_BLOCK
    grid = (n // t,)
    return pl.pallas_call(
        _finish_body,
        grid=grid,
        in_specs=[
            pl.BlockSpec((t * TOP_K, TOOL_EMB // 2), lambda i: (i, 0)),
            pl.BlockSpec((t * TOP_K, TOOL_EMB // 2), lambda i: (i, 0)),
        ],
        out_specs=pl.BlockSpec((t, TOP_K, TOOL_EMB), lambda i: (i, 0, 0)),
        out_shape=jax.ShapeDtypeStruct((n, TOP_K, TOOL_EMB), jnp.float32),
        compiler_params=pltpu.CompilerParams(
            dimension_semantics=("arbitrary",)),
    )(lo, hi)


def kernel(hidden_states, W_gate, b_gate, W_sel1, b_sel1, W_sel2, b_sel2,
           tool_table):
    b, s, hdim = hidden_states.shape
    n = b * s
    x2d = hidden_states.reshape(n, hdim)
    wg_t = W_gate.reshape(1, hdim)
    enh, probs, gate, idx, idx_sp = _run_tc(
        x2d, wg_t, b_gate.reshape(1, 1), W_sel1, b_sel1.reshape(1, TOOL_HID),
        W_sel2, b_sel2.reshape(1, MAX_TOOLS))
    half = TOOL_EMB // 2
    table_lo = jnp.tile(tool_table[:, :half], (TABLE_REP, 1))
    table_hi = jnp.tile(tool_table[:, half:], (TABLE_REP, 1))
    lo, hi = _sc_gather(table_lo, table_hi, idx_sp.reshape(n * TOP_K))
    embs = _tc_finish(lo, hi)
    return (
        enh.reshape(b, s, hdim),
        probs.reshape(b, s, MAX_TOOLS),
        gate.reshape(b, s, 1),
        idx.reshape(b, s, TOP_K),
        embs.reshape(b, s, TOP_K, TOOL_EMB),
    )


# speedup vs baseline: 1.2066x; 1.2066x over previous
import functools

import jax
import jax.numpy as jnp
from jax import lax
from jax.experimental import pallas as pl
from jax.experimental.pallas import tpu as pltpu
from jax.experimental.pallas import tpu_sc as plsc

HIDDEN = 2048
TOOL_HID = 512
MAX_TOOLS = 128
TOOL_EMB = 256
TOP_K = 3

TOKEN_BLOCK = 256

SC_CORES = 2
SC_SUBCORES = 16
SC_WORKERS = SC_CORES * SC_SUBCORES
GATHER_CHUNK = 128
TABLE_REP = 256


def _tc_body(x_ref, wgt_ref, bg_ref, w1_ref, b1_ref, w2_ref, b2_ref,
             enh_ref, probs_ref, gate_ref, idx_ref, idxs_ref):
    x = x_ref[...]
    enh_ref[...] = x

    xb = x.astype(jnp.bfloat16).astype(jnp.float32)
    wb = wgt_ref[...].astype(jnp.bfloat16).astype(jnp.float32)
    z = jnp.sum(xb * wb, axis=-1, keepdims=True) + bg_ref[...]
    gate_ref[...] = z > 0.0

    h = jnp.maximum(
        jnp.dot(x, w1_ref[...], preferred_element_type=jnp.float32)
        + b1_ref[...], 0.0)
    logits = (jnp.dot(h, w2_ref[...], preferred_element_type=jnp.float32)
              + b2_ref[...])
    m = jnp.max(logits, axis=-1, keepdims=True)
    e = jnp.exp(logits - m)
    probs = e / jnp.sum(e, axis=-1, keepdims=True)
    probs_ref[...] = probs

    iota = lax.broadcasted_iota(jnp.int32, probs.shape, 1)
    p = probs
    cols = []
    for _ in range(TOP_K):
        pm = jnp.max(p, axis=-1, keepdims=True)
        a = jnp.min(jnp.where(p == pm, iota, MAX_TOOLS), axis=-1,
                    keepdims=True)
        cols.append(a)
        p = jnp.where(iota == a, -1.0, p)
    idx = jnp.concatenate(cols, axis=1)
    idx_ref[...] = idx
    r3 = 3 * lax.broadcasted_iota(jnp.int32, idx.shape, 0)
    kk = lax.broadcasted_iota(jnp.int32, idx.shape, 1)
    idxs_ref[...] = idx + MAX_TOOLS * ((r3 + kk) % TABLE_REP)


def _run_tc(x2d, wg_t, b_gate, W_sel1, b_sel1, W_sel2, b_sel2):
    n = x2d.shape[0]
    grid = (n // TOKEN_BLOCK,)
    tok = lambda i: (i, 0)
    rep = lambda i: (0, 0)
    return pl.pallas_call(
        _tc_body,
        grid=grid,
        in_specs=[
            pl.BlockSpec((TOKEN_BLOCK, HIDDEN), tok),
            pl.BlockSpec((1, HIDDEN), rep),
            pl.BlockSpec((1, 1), rep),
            pl.BlockSpec((HIDDEN, TOOL_HID), rep),
            pl.BlockSpec((1, TOOL_HID), rep),
            pl.BlockSpec((TOOL_HID, MAX_TOOLS), rep),
            pl.BlockSpec((1, MAX_TOOLS), rep),
        ],
        out_specs=[
            pl.BlockSpec((TOKEN_BLOCK, HIDDEN), tok),
            pl.BlockSpec((TOKEN_BLOCK, MAX_TOOLS), tok),
            pl.BlockSpec((TOKEN_BLOCK, 1), tok),
            pl.BlockSpec((TOKEN_BLOCK, TOP_K), tok),
            pl.BlockSpec((TOKEN_BLOCK, TOP_K), tok),
        ],
        out_shape=[
            jax.ShapeDtypeStruct((n, HIDDEN), jnp.float32),
            jax.ShapeDtypeStruct((n, MAX_TOOLS), jnp.float32),
            jax.ShapeDtypeStruct((n, 1), jnp.bool_),
            jax.ShapeDtypeStruct((n, TOP_K), jnp.int32),
            jax.ShapeDtypeStruct((n, TOP_K), jnp.int32),
        ],
        compiler_params=pltpu.CompilerParams(
            dimension_semantics=("arbitrary",)),
    )(x2d, wg_t, b_gate, W_sel1, b_sel1, W_sel2, b_sel2)


def _sc_gather(table_lo, table_hi, idx_flat):
    total = idx_flat.shape[0]
    per_worker = total // SC_WORKERS
    chunks = per_worker // GATHER_CHUNK
    mesh = plsc.VectorSubcoreMesh(core_axis_name="c", subcore_axis_name="s")
    half = TOOL_EMB // 2
    L = 16

    @functools.partial(
        pl.kernel,
        out_type=[
            jax.ShapeDtypeStruct((total, half), jnp.float32),
            jax.ShapeDtypeStruct((total, half), jnp.float32),
        ],
        mesh=mesh,
        scratch_types=[
            pltpu.VMEM((per_worker,), jnp.int32),
            pltpu.VMEM((chunks, GATHER_CHUNK), jnp.int32),
            pltpu.VMEM((GATHER_CHUNK, half), jnp.float32),
            pltpu.VMEM((GATHER_CHUNK, half), jnp.float32),
            pltpu.VMEM((GATHER_CHUNK, half), jnp.float32),
            pltpu.VMEM((GATHER_CHUNK, half), jnp.float32),
            pltpu.SemaphoreType.DMA,
            pltpu.SemaphoreType.DMA,
            pltpu.SemaphoreType.DMA,
            pltpu.SemaphoreType.DMA,
            pltpu.SemaphoreType.DMA,
            pltpu.SemaphoreType.DMA,
            pltpu.SemaphoreType.DMA,
            pltpu.SemaphoreType.DMA,
        ],
    )
    def gather_kernel(tlo_hbm, thi_hbm, idx_hbm, lo_hbm, hi_hbm,
                      idx_all, dest_all, lo0, lo1, hi0, hi1,
                      sgl0, sgl1, sgh0, sgh1, ssl0, ssl1, ssh0, ssh1):
        wid = lax.axis_index("s") * SC_CORES + lax.axis_index("c")
        base = wid * per_worker
        pltpu.sync_copy(idx_hbm.at[pl.ds(base, per_worker)], idx_all)
        lane = lax.iota(jnp.int32, L)
        base3 = wid * (per_worker // 3)
        for j in range(chunks):
            off_j = j * GATHER_CHUNK
            r0 = off_j % 3
            t0 = base3 + off_j // 3
            for v in range(GATHER_CHUNK // L):
                su = v * L + r0 + lane
                tq = (su * 21846) >> 16
                t = t0 + tq
                k = su - 3 * tq
                bb = t >> 12
                ss_ = t & 4095
                dest_all[j, pl.ds(v * L, L)] = (bb * 12288 + k * 4096
                                                + ss_)
        lob = (lo0, lo1)
        hib = (hi0, hi1)
        sgl = (sgl0, sgl1)
        sgh = (sgh0, sgh1)
        ssl = (ssl0, ssl1)
        ssh = (ssh0, ssh1)
        gl, gh, sl, sh = {}, {}, {}, {}

        def idx_view(i):
            return idx_all.at[pl.ds(i * GATHER_CHUNK, GATHER_CHUNK)]

        gl[0] = pltpu.async_copy(tlo_hbm.at[idx_view(0)], lo0, sgl0)
        gh[0] = pltpu.async_copy(thi_hbm.at[idx_view(0)], hi0, sgh0)
        for i in range(chunks):
            bi = i & 1
            dv = dest_all.at[i]
            gl[i].wait()
            sl[i] = pltpu.async_copy(lob[bi], lo_hbm.at[dv], ssl[bi])
            gh[i].wait()
            sh[i] = pltpu.async_copy(hib[bi], hi_hbm.at[dv], ssh[bi])
            ni = i + 1
            if ni < chunks:
                nb = ni & 1
                if ni >= 2:
                    sl[ni - 2].wait()
                    sh[ni - 2].wait()
                gl[ni] = pltpu.async_copy(tlo_hbm.at[idx_view(ni)],
                                          lob[nb], sgl[nb])
                gh[ni] = pltpu.async_copy(thi_hbm.at[idx_view(ni)],
                                          hib[nb], sgh[nb])
        sl[chunks - 2].wait()
        sh[chunks - 2].wait()
        sl[chunks - 1].wait()
        sh[chunks - 1].wait()

    return gather_kernel(table_lo, table_hi, idx_flat)


def _finish_body(lo_ref, hi_ref, out_ref):
    lo = lo_ref[...]
    hi = hi_ref[...]
    out_ref[...] = jnp.concatenate([lo, hi], axis=-1)


def _tc_finish(lo4, hi4):
    b, k, s, half = lo4.shape
    sblk = TOKEN_BLOCK
    grid = (b * (s // sblk),)
    nsb = s // sblk
    return pl.pallas_call(
        _finish_body,
        grid=grid,
        in_specs=[
            pl.BlockSpec((1, k, sblk, half),
                         lambda i: (i // nsb, 0, i % nsb, 0)),
            pl.BlockSpec((1, k, sblk, half),
                         lambda i: (i // nsb, 0, i % nsb, 0)),
        ],
        out_specs=pl.BlockSpec((1, k, sblk, 2 * half),
                               lambda i: (i // nsb, 0, i % nsb, 0)),
        out_shape=jax.ShapeDtypeStruct((b, k, s, 2 * half), jnp.float32),
        compiler_params=pltpu.CompilerParams(
            dimension_semantics=("arbitrary",)),
    )(lo4, hi4)


def kernel(hidden_states, W_gate, b_gate, W_sel1, b_sel1, W_sel2, b_sel2,
           tool_table):
    b, s, hdim = hidden_states.shape
    n = b * s
    x2d = hidden_states.reshape(n, hdim)
    wg_t = W_gate.reshape(1, hdim)
    enh, probs, gate, idx, idx_sp = _run_tc(
        x2d, wg_t, b_gate.reshape(1, 1), W_sel1, b_sel1.reshape(1, TOOL_HID),
        W_sel2, b_sel2.reshape(1, MAX_TOOLS))
    half = TOOL_EMB // 2
    table_lo = jnp.tile(tool_table[:, :half], (TABLE_REP, 1))
    table_hi = jnp.tile(tool_table[:, half:], (TABLE_REP, 1))
    lo, hi = _sc_gather(table_lo, table_hi, idx_sp.reshape(n * TOP_K))
    lo4 = lo.reshape(b, TOP_K, s, half)
    hi4 = hi.reshape(b, TOP_K, s, half)
    embs = _tc_finish(lo4, hi4).transpose(0, 2, 1, 3)
    return (
        enh.reshape(b, s, hdim),
        probs.reshape(b, s, MAX_TOOLS),
        gate.reshape(b, s, 1),
        idx.reshape(b, s, TOP_K),
        embs,
    )
